# baseline (speedup 1.0000x reference)
import functools

import jax
import jax.numpy as jnp
import numpy as np
from jax import lax
from jax.experimental import pallas as pl
from jax.experimental.pallas import tpu as pltpu
from jax.experimental.pallas import tpu_sc as plsc

VOCAB = 1000000
EMB = 64
B = 16384
L = 200
NCLS = 20

NC = 2
NS = 16
LANES = 16
NW = NC * NS
ROWS_PER_W = B // NW
EW = EMB // 2
NVEC = EMB // LANES
C0, C1 = 104, 96
RPP = 128
NPH = ROWS_PER_W // RPP
NRING = 4
UNROLL = 8

_PERM = np.array(
    [32 * (c // 32) + 2 * (c % 16) + (c // 16) % 2 for c in range(EMB)]
)


def _sc_pool(data_flat, table_i32):
    mesh = plsc.VectorSubcoreMesh(
        core_axis_name="c", subcore_axis_name="s", num_cores=NC, num_subcores=NS
    )

    @functools.partial(
        pl.kernel,
        out_type=jax.ShapeDtypeStruct((B, EMB), jnp.float32),
        mesh=mesh,
        compiler_params=pltpu.CompilerParams(use_tc_tiling_on_sc=False),
        scratch_types=[
            pltpu.VMEM((2, RPP * L), jnp.int32),
            pltpu.VMEM((NRING, L, EW), jnp.int32),
            pltpu.VMEM((2, RPP, EMB), jnp.float32),
            pltpu.SemaphoreType.DMA,
            pltpu.SemaphoreType.DMA,
            pltpu.SemaphoreType.DMA,
            pltpu.SemaphoreType.DMA,
            pltpu.SemaphoreType.DMA,
            pltpu.SemaphoreType.DMA,
            pltpu.SemaphoreType.DMA,
            pltpu.SemaphoreType.DMA,
        ],
    )
    def k(data_hbm, table_hbm, out_hbm, idxg, rows, ostage,
          is0, is1, g0, g1, g2, g3, o0, o1):
        isem = (is0, is1)
        gsem = (g0, g1, g2, g3)
        osem = (o0, o1)
        wid = lax.axis_index("s") * NC + lax.axis_index("c")
        base = wid * ROWS_PER_W

        def issue_idx(p, pp):
            return pltpu.async_copy(
                data_hbm.at[pl.ds((base + p * RPP) * L, RPP * L)],
                idxg.at[pp], isem[pp])

        def issue_gathers(idx_p, roff, slot):
            off = roff * L
            pltpu.async_copy(
                table_hbm.at[idx_p.at[pl.ds(off, C0)]],
                rows.at[slot].at[pl.ds(0, C0)], gsem[slot])
            pltpu.async_copy(
                table_hbm.at[idx_p.at[pl.ds(off + C0, C1)]],
                rows.at[slot].at[pl.ds(C0, C1)], gsem[slot])

        def wait_gathers(slot):
            pltpu.make_async_copy(
                table_hbm.at[pl.ds(0, L)], rows.at[slot], gsem[slot]).wait()

        def accumulate(slot):
            slot_ref = rows.at[slot]

            def body(jj, accs):
                accs = list(accs)
                for u in range(UNROLL):
                    j = jj * UNROLL + u
                    for g in range(2):
                        w = slot_ref[j, pl.ds(g * LANES, LANES)]
                        even = lax.bitcast_convert_type(w << 16, jnp.float32)
                        odd = lax.bitcast_convert_type(
                            lax.shift_right_logical(w, 16) << 16, jnp.float32)
                        accs[2 * g] = accs[2 * g] + even
                        accs[2 * g + 1] = accs[2 * g + 1] + odd
                return tuple(accs)

            accs = tuple(jnp.zeros((LANES,), jnp.float32) for _ in range(NVEC))
            return lax.fori_loop(0, L // UNROLL, body, accs)

        def store_row(opp, r, accs):
            for t in range(NVEC):
                opp[r, pl.ds(t * LANES, LANES)] = accs[t]

        idesc = [issue_idx(0, 0), None]
        odesc = [None, None]
        for p in range(NPH):
            pp = p % 2
            if odesc[pp] is not None:
                odesc[pp].wait()
            idesc[pp].wait()
            if p + 1 < NPH:
                idesc[(p + 1) % 2] = issue_idx(p + 1, (p + 1) % 2)
            idx_p = idxg.at[pp]
            opp = ostage.at[pp]
            for s in range(NRING):
                issue_gathers(idx_p, s, s)

            def inner(h, carry, idx_p=idx_p, opp=opp):
                for j in range(NRING):
                    r = NRING * h + j
                    wait_gathers(j)
                    accs = accumulate(j)
                    store_row(opp, r, accs)
                    issue_gathers(idx_p, r + NRING, j)
                return carry

            lax.fori_loop(0, RPP // NRING - 1, inner, 0)
            for j in range(NRING):
                r = RPP - NRING + j
                wait_gathers(j)
                accs = accumulate(j)
                store_row(opp, r, accs)
            odesc[pp] = pltpu.async_copy(
                opp, out_hbm.at[pl.ds(base + p * RPP, RPP)], osem[pp])
        odesc[0].wait()
        odesc[1].wait()

    return k(data_flat, table_i32)


def _tc_linear(sums, inv_len, W2, b2):
    BLK = 2048

    def body(s_ref, l_ref, w_ref, b_ref, o_ref):
        pooled = s_ref[...] * l_ref[...]
        o_ref[...] = (
            lax.dot_general(
                pooled, w_ref[...], (((1,), (1,)), ((), ())),
                preferred_element_type=jnp.float32,
            )
            + b_ref[...]
        )

    return pl.pallas_call(
        body,
        grid=(B // BLK,),
        in_specs=[
            pl.BlockSpec((BLK, EMB), lambda i: (i, 0)),
            pl.BlockSpec((BLK, 1), lambda i: (i, 0)),
            pl.BlockSpec((NCLS, EMB), lambda i: (0, 0)),
            pl.BlockSpec((1, NCLS), lambda i: (0, 0)),
        ],
        out_specs=pl.BlockSpec((BLK, NCLS), lambda i: (i, 0)),
        out_shape=jax.ShapeDtypeStruct((B, NCLS), jnp.float32),
    )(sums, inv_len, W2, b2)


def kernel(data, length, embed_table, W, b):
    data_flat = data.reshape(B * L).astype(jnp.int32)
    table_i32 = lax.bitcast_convert_type(
        embed_table.astype(jnp.bfloat16).reshape(VOCAB, EW, 2), jnp.int32)
    sums = _sc_pool(data_flat, table_i32)
    inv_len = (1.0 / length.astype(jnp.float32)).reshape(B, 1)
    W_perm = W[:, _PERM]
    return _tc_linear(sums, inv_len, W_perm, b.reshape(1, NCLS))


# speedup vs baseline: 1.2290x; 1.2290x over previous
import functools

import jax
import jax.numpy as jnp
import numpy as np
from jax import lax
from jax.experimental import pallas as pl
from jax.experimental.pallas import tpu as pltpu
from jax.experimental.pallas import tpu_sc as plsc

VOCAB = 1000000
EMB = 64
B = 16384
L = 200
NCLS = 20

NC = 2
NS = 16
LANES = 16
NW = NC * NS
ROWS_PER_W = B // NW
EW = EMB // 2
NVEC = EMB // LANES
C0, C1 = 104, 96
RPP = 128
NPH = ROWS_PER_W // RPP
NRING = 4
UNROLL = 8

_PERM = np.array(
    [32 * ((c // 16) % 2) + 16 * (c // 32) + (c % 16) for c in range(EMB)]
)


def _sc_pool(data_flat, table_i32):
    mesh = plsc.VectorSubcoreMesh(
        core_axis_name="c", subcore_axis_name="s", num_cores=NC, num_subcores=NS
    )

    @functools.partial(
        pl.kernel,
        out_type=jax.ShapeDtypeStruct((B, EMB), jnp.float32),
        mesh=mesh,
        compiler_params=pltpu.CompilerParams(use_tc_tiling_on_sc=False),
        scratch_types=[
            pltpu.VMEM((2, RPP * L), jnp.int32),
            pltpu.VMEM((NRING, L, EW), jnp.int32),
            pltpu.VMEM((2, RPP, EMB), jnp.float32),
            pltpu.SemaphoreType.DMA,
            pltpu.SemaphoreType.DMA,
            pltpu.SemaphoreType.DMA,
            pltpu.SemaphoreType.DMA,
            pltpu.SemaphoreType.DMA,
            pltpu.SemaphoreType.DMA,
            pltpu.SemaphoreType.DMA,
            pltpu.SemaphoreType.DMA,
        ],
    )
    def k(data_hbm, table_hbm, out_hbm, idxg, rows, ostage,
          is0, is1, g0, g1, g2, g3, o0, o1):
        isem = (is0, is1)
        gsem = (g0, g1, g2, g3)
        osem = (o0, o1)
        wid = lax.axis_index("s") * NC + lax.axis_index("c")
        base = wid * ROWS_PER_W

        def issue_idx(p, pp):
            return pltpu.async_copy(
                data_hbm.at[pl.ds((base + p * RPP) * L, RPP * L)],
                idxg.at[pp], isem[pp])

        def issue_gathers(idx_p, roff, slot):
            off = roff * L
            pltpu.async_copy(
                table_hbm.at[idx_p.at[pl.ds(off, C0)]],
                rows.at[slot].at[pl.ds(0, C0)], gsem[slot])
            pltpu.async_copy(
                table_hbm.at[idx_p.at[pl.ds(off + C0, C1)]],
                rows.at[slot].at[pl.ds(C0, C1)], gsem[slot])

        def wait_gathers(slot):
            pltpu.make_async_copy(
                table_hbm.at[pl.ds(0, L)], rows.at[slot], gsem[slot]).wait()

        def accumulate(slot):
            slot_ref = rows.at[slot]

            def body(jj, accs):
                accs = list(accs)
                for u in range(UNROLL):
                    j = jj * UNROLL + u
                    for g in range(2):
                        w = slot_ref[j, pl.ds(g * LANES, LANES)]
                        even = lax.bitcast_convert_type(w << 16, jnp.float32)
                        odd = lax.bitcast_convert_type(
                            lax.shift_right_logical(w, 16) << 16, jnp.float32)
                        accs[2 * g] = accs[2 * g] + even
                        accs[2 * g + 1] = accs[2 * g + 1] + odd
                return tuple(accs)

            accs = tuple(jnp.zeros((LANES,), jnp.float32) for _ in range(NVEC))
            return lax.fori_loop(0, L // UNROLL, body, accs)

        def store_row(opp, r, accs):
            for t in range(NVEC):
                opp[r, pl.ds(t * LANES, LANES)] = accs[t]

        idesc = [issue_idx(0, 0), None]
        odesc = [None, None]
        for p in range(NPH):
            pp = p % 2
            if odesc[pp] is not None:
                odesc[pp].wait()
            idesc[pp].wait()
            if p + 1 < NPH:
                idesc[(p + 1) % 2] = issue_idx(p + 1, (p + 1) % 2)
            idx_p = idxg.at[pp]
            opp = ostage.at[pp]
            for s in range(NRING):
                issue_gathers(idx_p, s, s)

            def inner(h, carry, idx_p=idx_p, opp=opp):
                for j in range(NRING):
                    r = NRING * h + j
                    wait_gathers(j)
                    accs = accumulate(j)
                    store_row(opp, r, accs)
                    issue_gathers(idx_p, r + NRING, j)
                return carry

            lax.fori_loop(0, RPP // NRING - 1, inner, 0)
            for j in range(NRING):
                r = RPP - NRING + j
                wait_gathers(j)
                accs = accumulate(j)
                store_row(opp, r, accs)
            odesc[pp] = pltpu.async_copy(
                opp, out_hbm.at[pl.ds(base + p * RPP, RPP)], osem[pp])
        odesc[0].wait()
        odesc[1].wait()

    return k(data_flat, table_i32)


def _tc_linear(sums, inv_len, W2, b2):
    BLK = 2048

    def body(s_ref, l_ref, w_ref, b_ref, o_ref):
        pooled = s_ref[...] * l_ref[...]
        o_ref[...] = (
            lax.dot_general(
                pooled, w_ref[...], (((1,), (1,)), ((), ())),
                preferred_element_type=jnp.float32,
            )
            + b_ref[...]
        )

    return pl.pallas_call(
        body,
        grid=(B // BLK,),
        in_specs=[
            pl.BlockSpec((BLK, EMB), lambda i: (i, 0)),
            pl.BlockSpec((BLK, 1), lambda i: (i, 0)),
            pl.BlockSpec((NCLS, EMB), lambda i: (0, 0)),
            pl.BlockSpec((1, NCLS), lambda i: (0, 0)),
        ],
        out_specs=pl.BlockSpec((BLK, NCLS), lambda i: (i, 0)),
        out_shape=jax.ShapeDtypeStruct((B, NCLS), jnp.float32),
    )(sums, inv_len, W2, b2)


def kernel(data, length, embed_table, W, b):
    data_flat = data.reshape(B * L).astype(jnp.int32)
    u = lax.bitcast_convert_type(embed_table, jnp.uint32)

    def _rne(x):
        return (x + jnp.uint32(0x7FFF) + ((x >> 16) & jnp.uint32(1))) >> 16

    packed = _rne(u[:, :EW]) | (_rne(u[:, EW:]) << 16)
    table_i32 = lax.bitcast_convert_type(packed, jnp.int32)
    sums = _sc_pool(data_flat, table_i32)
    inv_len = (1.0 / length.astype(jnp.float32)).reshape(B, 1)
    W_perm = W[:, _PERM]
    return _tc_linear(sums, inv_len, W_perm, b.reshape(1, NCLS))


# speedup vs baseline: 1.5485x; 1.2599x over previous
import functools

import jax
import jax.numpy as jnp
import numpy as np
from jax import lax
from jax.experimental import pallas as pl
from jax.experimental.pallas import tpu as pltpu
from jax.experimental.pallas import tpu_sc as plsc

VOCAB = 1000000
EMB = 64
B = 16384
L = 200
NCLS = 20

NC = 2
NS = 16
LANES = 16
NW = NC * NS
ROWS_PER_W = B // NW
EW = EMB // 2
NVEC = EMB // LANES
C0, C1 = 104, 96
RPP = 128
NPH = ROWS_PER_W // RPP
NRING = 4
UNROLL = 8

_PERM = np.array(
    [32 * ((c // 16) % 2) + 16 * (c // 32) + (c % 16) for c in range(EMB)]
)


def _sc_pool(data_flat, table_i32):
    mesh = plsc.VectorSubcoreMesh(
        core_axis_name="c", subcore_axis_name="s", num_cores=NC, num_subcores=NS
    )

    @functools.partial(
        pl.kernel,
        out_type=jax.ShapeDtypeStruct((B, EMB), jnp.float32),
        mesh=mesh,
        compiler_params=pltpu.CompilerParams(use_tc_tiling_on_sc=False),
        scratch_types=[
            pltpu.VMEM((2, RPP * L), jnp.int32),
            pltpu.VMEM((NRING, L, EW), jnp.int32),
            pltpu.VMEM((2, RPP, EMB), jnp.float32),
            pltpu.SemaphoreType.DMA,
            pltpu.SemaphoreType.DMA,
            pltpu.SemaphoreType.DMA,
            pltpu.SemaphoreType.DMA,
            pltpu.SemaphoreType.DMA,
            pltpu.SemaphoreType.DMA,
            pltpu.SemaphoreType.DMA,
            pltpu.SemaphoreType.DMA,
        ],
    )
    def k(data_hbm, table_hbm, out_hbm, idxg, rows, ostage,
          is0, is1, g0, g1, g2, g3, o0, o1):
        isem = (is0, is1)
        gsem = (g0, g1, g2, g3)
        osem = (o0, o1)
        wid = lax.axis_index("s") * NC + lax.axis_index("c")
        base = wid * ROWS_PER_W

        def issue_idx(p, pp):
            return pltpu.async_copy(
                data_hbm.at[pl.ds((base + p * RPP) * L, RPP * L)],
                idxg.at[pp], isem[pp])

        def issue_gathers(idx_p, roff, slot):
            off = roff * L
            pltpu.async_copy(
                table_hbm.at[idx_p.at[pl.ds(off, C0)]],
                rows.at[slot].at[pl.ds(0, C0)], gsem[slot])
            pltpu.async_copy(
                table_hbm.at[idx_p.at[pl.ds(off + C0, C1)]],
                rows.at[slot].at[pl.ds(C0, C1)], gsem[slot])

        def wait_gathers(slot):
            pltpu.make_async_copy(
                table_hbm.at[pl.ds(0, L)], rows.at[slot], gsem[slot]).wait()

        def accumulate(slot):
            slot_ref = rows.at[slot]

            def body(jj, accs):
                accs = list(accs)
                for u in range(UNROLL):
                    j = jj * UNROLL + u
                    for g in range(2):
                        w = slot_ref[j, pl.ds(g * LANES, LANES)]
                        even = lax.bitcast_convert_type(w << 16, jnp.float32)
                        odd = lax.bitcast_convert_type(
                            lax.shift_right_logical(w, 16) << 16, jnp.float32)
                        accs[2 * g] = accs[2 * g] + even
                        accs[2 * g + 1] = accs[2 * g + 1] + odd
                return tuple(accs)

            accs = tuple(jnp.zeros((LANES,), jnp.float32) for _ in range(NVEC))
            return lax.fori_loop(0, L // UNROLL, body, accs)

        def store_row(opp, r, accs):
            for t in range(NVEC):
                opp[r, pl.ds(t * LANES, LANES)] = accs[t]

        idesc = [issue_idx(0, 0), None]
        odesc = [None, None]
        for p in range(NPH):
            pp = p % 2
            if odesc[pp] is not None:
                odesc[pp].wait()
            idesc[pp].wait()
            if p + 1 < NPH:
                idesc[(p + 1) % 2] = issue_idx(p + 1, (p + 1) % 2)
            idx_p = idxg.at[pp]
            opp = ostage.at[pp]
            for s in range(NRING):
                issue_gathers(idx_p, s, s)

            def inner(h, carry, idx_p=idx_p, opp=opp):
                for j in range(NRING):
                    r = NRING * h + j
                    wait_gathers(j)
                    accs = accumulate(j)
                    store_row(opp, r, accs)
                    issue_gathers(idx_p, r + NRING, j)
                return carry

            lax.fori_loop(0, RPP // NRING - 1, inner, 0)
            for j in range(NRING):
                r = RPP - NRING + j
                wait_gathers(j)
                accs = accumulate(j)
                store_row(opp, r, accs)
            odesc[pp] = pltpu.async_copy(
                opp, out_hbm.at[pl.ds(base + p * RPP, RPP)], osem[pp])
        odesc[0].wait()
        odesc[1].wait()

    return k(data_flat, table_i32)


def _tc_linear(sums, inv_len, W2, b2):
    BLK = 2048

    def body(s_ref, l_ref, w_ref, b_ref, o_ref):
        pooled = s_ref[...] * l_ref[...]
        o_ref[...] = (
            lax.dot_general(
                pooled, w_ref[...], (((1,), (1,)), ((), ())),
                preferred_element_type=jnp.float32,
            )
            + b_ref[...]
        )

    return pl.pallas_call(
        body,
        grid=(B // BLK,),
        in_specs=[
            pl.BlockSpec((BLK, EMB), lambda i: (i, 0)),
            pl.BlockSpec((BLK, 1), lambda i: (i, 0)),
            pl.BlockSpec((NCLS, EMB), lambda i: (0, 0)),
            pl.BlockSpec((1, NCLS), lambda i: (0, 0)),
        ],
        out_specs=pl.BlockSpec((BLK, NCLS), lambda i: (i, 0)),
        out_shape=jax.ShapeDtypeStruct((B, NCLS), jnp.float32),
    )(sums, inv_len, W2, b2)


def _tc_pack(table):
    BLKV = 10000

    def body(x_ref, o_ref):
        u = lax.bitcast_convert_type(x_ref[...], jnp.uint32)

        def rne(x):
            return (x + jnp.uint32(0x7FFF) + ((x >> 16) & jnp.uint32(1))) >> 16

        o_ref[...] = lax.bitcast_convert_type(
            rne(u[:, :EW]) | (rne(u[:, EW:]) << 16), jnp.int32)

    return pl.pallas_call(
        body,
        grid=(VOCAB // BLKV,),
        in_specs=[pl.BlockSpec((BLKV, EMB), lambda i: (i, 0))],
        out_specs=pl.BlockSpec((BLKV, EW), lambda i: (i, 0)),
        out_shape=jax.ShapeDtypeStruct((VOCAB, EW), jnp.int32),
    )(table)


def kernel(data, length, embed_table, W, b):
    data_flat = data.reshape(B * L).astype(jnp.int32)
    table_i32 = _tc_pack(embed_table)
    sums = _sc_pool(data_flat, table_i32)
    inv_len = (1.0 / length.astype(jnp.float32)).reshape(B, 1)
    W_perm = W[:, _PERM]
    return _tc_linear(sums, inv_len, W_perm, b.reshape(1, NCLS))


# speedup vs baseline: 1.5901x; 1.0269x over previous
import functools

import jax
import jax.numpy as jnp
import numpy as np
from jax import lax
from jax.experimental import pallas as pl
from jax.experimental.pallas import tpu as pltpu
from jax.experimental.pallas import tpu_sc as plsc

VOCAB = 1000000
EMB = 64
B = 16384
L = 200
NCLS = 20

NC = 2
NS = 16
LANES = 16
NW = NC * NS
ROWS_PER_W = B // NW
EW = EMB // 2
NVEC = EMB // LANES
C0, C1 = 104, 96
RPP = 128
NPH = ROWS_PER_W // RPP
NRING = 4
UNROLL = 8

_PERM = np.array(
    [32 * ((c // 16) % 2) + 16 * (c // 32) + (c % 16) for c in range(EMB)]
)


def _sc_pool(data_flat, table_i32):
    mesh = plsc.VectorSubcoreMesh(
        core_axis_name="c", subcore_axis_name="s", num_cores=NC, num_subcores=NS
    )

    @functools.partial(
        pl.kernel,
        out_type=jax.ShapeDtypeStruct((B, EMB), jnp.float32),
        mesh=mesh,
        compiler_params=pltpu.CompilerParams(use_tc_tiling_on_sc=False),
        scratch_types=[
            pltpu.VMEM((2, RPP * L), jnp.int32),
            pltpu.VMEM((NRING, L, EW), jnp.int32),
            pltpu.VMEM((2, RPP, EMB), jnp.float32),
            pltpu.SemaphoreType.DMA,
            pltpu.SemaphoreType.DMA,
            pltpu.SemaphoreType.DMA,
            pltpu.SemaphoreType.DMA,
            pltpu.SemaphoreType.DMA,
            pltpu.SemaphoreType.DMA,
            pltpu.SemaphoreType.DMA,
            pltpu.SemaphoreType.DMA,
        ],
    )
    def k(data_hbm, table_hbm, out_hbm, idxg, rows, ostage,
          is0, is1, g0, g1, g2, g3, o0, o1):
        isem = (is0, is1)
        gsem = (g0, g1, g2, g3)
        osem = (o0, o1)
        wid = lax.axis_index("s") * NC + lax.axis_index("c")
        base = wid * ROWS_PER_W

        def issue_idx(p, pp):
            return pltpu.async_copy(
                data_hbm.at[pl.ds((base + p * RPP) * L, RPP * L)],
                idxg.at[pp], isem[pp])

        def issue_gathers(idx_p, roff, slot):
            off = roff * L
            pltpu.async_copy(
                table_hbm.at[idx_p.at[pl.ds(off, C0)]],
                rows.at[slot].at[pl.ds(0, C0)], gsem[slot])
            pltpu.async_copy(
                table_hbm.at[idx_p.at[pl.ds(off + C0, C1)]],
                rows.at[slot].at[pl.ds(C0, C1)], gsem[slot])

        def wait_gathers(slot):
            pltpu.make_async_copy(
                table_hbm.at[pl.ds(0, L)], rows.at[slot], gsem[slot]).wait()

        def accumulate(slot):
            slot_ref = rows.at[slot]

            def body(jj, accs):
                accs = list(accs)
                for u in range(UNROLL):
                    j = jj * UNROLL + u
                    for g in range(2):
                        w = slot_ref[j, pl.ds(g * LANES, LANES)]
                        even = lax.bitcast_convert_type(w << 16, jnp.float32)
                        odd = lax.bitcast_convert_type(
                            lax.shift_right_logical(w, 16) << 16, jnp.float32)
                        accs[2 * g] = accs[2 * g] + even
                        accs[2 * g + 1] = accs[2 * g + 1] + odd
                return tuple(accs)

            accs = tuple(jnp.zeros((LANES,), jnp.float32) for _ in range(NVEC))
            return lax.fori_loop(0, L // UNROLL, body, accs)

        def store_row(opp, r, accs):
            for t in range(NVEC):
                opp[r, pl.ds(t * LANES, LANES)] = accs[t]

        idesc = [issue_idx(0, 0), None]
        odesc = [None, None]
        for p in range(NPH):
            pp = p % 2
            if odesc[pp] is not None:
                odesc[pp].wait()
            idesc[pp].wait()
            if p + 1 < NPH:
                idesc[(p + 1) % 2] = issue_idx(p + 1, (p + 1) % 2)
            idx_p = idxg.at[pp]
            opp = ostage.at[pp]
            for s in range(NRING):
                issue_gathers(idx_p, s, s)

            def inner(h, carry, idx_p=idx_p, opp=opp):
                for j in range(NRING):
                    r = NRING * h + j
                    wait_gathers(j)
                    accs = accumulate(j)
                    store_row(opp, r, accs)
                    issue_gathers(idx_p, r + NRING, j)
                return carry

            lax.fori_loop(0, RPP // NRING - 1, inner, 0)
            for j in range(NRING):
                r = RPP - NRING + j
                wait_gathers(j)
                accs = accumulate(j)
                store_row(opp, r, accs)
            odesc[pp] = pltpu.async_copy(
                opp, out_hbm.at[pl.ds(base + p * RPP, RPP)], osem[pp])
        odesc[0].wait()
        odesc[1].wait()

    return k(data_flat, table_i32)


def _tc_linear(sums, inv_len, W2, b2):
    BLK = 2048

    def body(s_ref, l_ref, w_ref, b_ref, o_ref):
        pooled = s_ref[...] * l_ref[...]
        o_ref[...] = (
            lax.dot_general(
                pooled, w_ref[...], (((1,), (1,)), ((), ())),
                preferred_element_type=jnp.float32,
            )
            + b_ref[...]
        )

    return pl.pallas_call(
        body,
        grid=(B // BLK,),
        in_specs=[
            pl.BlockSpec((BLK, EMB), lambda i: (i, 0)),
            pl.BlockSpec((BLK, 1), lambda i: (i, 0)),
            pl.BlockSpec((NCLS, EMB), lambda i: (0, 0)),
            pl.BlockSpec((1, NCLS), lambda i: (0, 0)),
        ],
        out_specs=pl.BlockSpec((BLK, NCLS), lambda i: (i, 0)),
        out_shape=jax.ShapeDtypeStruct((B, NCLS), jnp.float32),
    )(sums, inv_len, W2, b2)


def _tc_pack(table128):
    VH = VOCAB // 2
    BLKV = 10000

    def body(x_ref, o_ref):
        u = lax.bitcast_convert_type(x_ref[...], jnp.uint32)

        def rne(x):
            return (x + jnp.uint32(0x7FFF) + ((x >> 16) & jnp.uint32(1))) >> 16

        a = rne(u[:, 0:EW]) | (rne(u[:, EW:EMB]) << 16)
        c = rne(u[:, EMB:EMB + EW]) | (rne(u[:, EMB + EW:]) << 16)
        o_ref[...] = lax.bitcast_convert_type(
            jnp.concatenate([a, c], axis=1), jnp.int32)

    return pl.pallas_call(
        body,
        grid=(VH // BLKV,),
        in_specs=[pl.BlockSpec((BLKV, 2 * EMB), lambda i: (i, 0))],
        out_specs=pl.BlockSpec((BLKV, EMB), lambda i: (i, 0)),
        out_shape=jax.ShapeDtypeStruct((VH, EMB), jnp.int32),
    )(table128)


def kernel(data, length, embed_table, W, b):
    data_flat = data.reshape(B * L).astype(jnp.int32)
    table_i32 = _tc_pack(
        embed_table.reshape(VOCAB // 2, 2 * EMB)).reshape(VOCAB, EW)
    sums = _sc_pool(data_flat, table_i32)
    inv_len = (1.0 / length.astype(jnp.float32)).reshape(B, 1)
    W_perm = W[:, _PERM]
    return _tc_linear(sums, inv_len, W_perm, b.reshape(1, NCLS))


# speedup vs baseline: 2.0648x; 1.2985x over previous
import functools

import jax
import jax.numpy as jnp
from jax import lax
from jax.experimental import pallas as pl
from jax.experimental.pallas import tpu as pltpu
from jax.experimental.pallas import tpu_sc as plsc

VOCAB = 1000000
EMB = 64
B = 16384
L = 200
NCLS = 20

NC = 2
NS = 16
LANES = 16
NW = NC * NS
ROWS_PER_W = B // NW
C0, C1 = 104, 96
NVEC = EMB // LANES
RPP = 128
NPH = ROWS_PER_W // RPP
NRING = 4
UNROLL = 8


def _sc_pool(data_flat, table):
    mesh = plsc.VectorSubcoreMesh(
        core_axis_name="c", subcore_axis_name="s", num_cores=NC, num_subcores=NS
    )

    @functools.partial(
        pl.kernel,
        out_type=jax.ShapeDtypeStruct((B, EMB), jnp.float32),
        mesh=mesh,
        compiler_params=pltpu.CompilerParams(use_tc_tiling_on_sc=False),
        scratch_types=[
            pltpu.VMEM((2, RPP * L), jnp.int32),
            pltpu.VMEM((NRING, L, EMB), jnp.float32),
            pltpu.VMEM((2, RPP, EMB), jnp.float32),
            pltpu.SemaphoreType.DMA,
            pltpu.SemaphoreType.DMA,
            pltpu.SemaphoreType.DMA,
            pltpu.SemaphoreType.DMA,
            pltpu.SemaphoreType.DMA,
            pltpu.SemaphoreType.DMA,
            pltpu.SemaphoreType.DMA,
            pltpu.SemaphoreType.DMA,
        ],
    )
    def k(data_hbm, table_hbm, out_hbm, idxg, rows, ostage,
          is0, is1, g0, g1, g2, g3, o0, o1):
        isem = (is0, is1)
        gsem = (g0, g1, g2, g3)
        osem = (o0, o1)
        wid = lax.axis_index("s") * NC + lax.axis_index("c")
        base = wid * ROWS_PER_W

        def issue_idx(p, pp):
            return pltpu.async_copy(
                data_hbm.at[pl.ds((base + p * RPP) * L, RPP * L)],
                idxg.at[pp], isem[pp])

        def issue_gathers(idx_p, roff, slot):
            off = roff * L
            pltpu.async_copy(
                table_hbm.at[idx_p.at[pl.ds(off, C0)]],
                rows.at[slot].at[pl.ds(0, C0)], gsem[slot])
            pltpu.async_copy(
                table_hbm.at[idx_p.at[pl.ds(off + C0, C1)]],
                rows.at[slot].at[pl.ds(C0, C1)], gsem[slot])

        def wait_gathers(slot):
            pltpu.make_async_copy(
                table_hbm.at[pl.ds(0, L)], rows.at[slot], gsem[slot]).wait()

        def accumulate(slot):
            slot_ref = rows.at[slot]

            def body(jj, accs):
                accs = list(accs)
                for u in range(UNROLL):
                    j = jj * UNROLL + u
                    for t in range(NVEC):
                        accs[t] = accs[t] + slot_ref[j, pl.ds(t * LANES, LANES)]
                return tuple(accs)

            accs = tuple(jnp.zeros((LANES,), jnp.float32) for _ in range(NVEC))
            return lax.fori_loop(0, L // UNROLL, body, accs)

        def store_row(opp, r, accs):
            for t in range(NVEC):
                opp[r, pl.ds(t * LANES, LANES)] = accs[t]

        idesc = [issue_idx(0, 0), None]
        odesc = [None, None]
        for p in range(NPH):
            pp = p % 2
            if odesc[pp] is not None:
                odesc[pp].wait()
            idesc[pp].wait()
            if p + 1 < NPH:
                idesc[(p + 1) % 2] = issue_idx(p + 1, (p + 1) % 2)
            idx_p = idxg.at[pp]
            opp = ostage.at[pp]
            for s in range(NRING):
                issue_gathers(idx_p, s, s)

            def inner(h, carry, idx_p=idx_p, opp=opp):
                for j in range(NRING):
                    r = NRING * h + j
                    wait_gathers(j)
                    accs = accumulate(j)
                    store_row(opp, r, accs)
                    issue_gathers(idx_p, r + NRING, j)
                return carry

            lax.fori_loop(0, RPP // NRING - 1, inner, 0)
            for j in range(NRING):
                r = RPP - NRING + j
                wait_gathers(j)
                accs = accumulate(j)
                store_row(opp, r, accs)
            odesc[pp] = pltpu.async_copy(
                opp, out_hbm.at[pl.ds(base + p * RPP, RPP)], osem[pp])
        odesc[0].wait()
        odesc[1].wait()

    return k(data_flat, table)


def _tc_linear(sums, inv_len, W2, b2):
    BLK = 2048

    def body(s_ref, l_ref, w_ref, b_ref, o_ref):
        pooled = s_ref[...] * l_ref[...]
        o_ref[...] = (
            lax.dot_general(
                pooled, w_ref[...], (((1,), (1,)), ((), ())),
                preferred_element_type=jnp.float32,
            )
            + b_ref[...]
        )

    return pl.pallas_call(
        body,
        grid=(B // BLK,),
        in_specs=[
            pl.BlockSpec((BLK, EMB), lambda i: (i, 0)),
            pl.BlockSpec((BLK, 1), lambda i: (i, 0)),
            pl.BlockSpec((NCLS, EMB), lambda i: (0, 0)),
            pl.BlockSpec((1, NCLS), lambda i: (0, 0)),
        ],
        out_specs=pl.BlockSpec((BLK, NCLS), lambda i: (i, 0)),
        out_shape=jax.ShapeDtypeStruct((B, NCLS), jnp.float32),
    )(sums, inv_len, W2, b2)


def kernel(data, length, embed_table, W, b):
    data_flat = data.reshape(B * L).astype(jnp.int32)
    sums = _sc_pool(data_flat, embed_table)
    inv_len = (1.0 / length.astype(jnp.float32)).reshape(B, 1)
    return _tc_linear(sums, inv_len, W, b.reshape(1, NCLS))
